# baseline (speedup 1.0000x reference)
import functools

import jax
import jax.numpy as jnp
from jax import lax
from jax.experimental import pallas as pl
from jax.experimental.pallas import tpu as pltpu
from jax.experimental.pallas import tpu_sc as plsc

EMBED = 128
RPC = 2
NB = 4
GD = 2
SD = 2


@functools.partial(jax.jit, static_argnums=(2, 3, 4, 5))
def _sc_gather(table, idx3, nw, k_per_w, b, s):
    mesh = plsc.VectorSubcoreMesh(core_axis_name="c", subcore_axis_name="s")
    idxc = idx3.shape[-1]
    rows_per_w = b // nw
    assert k_per_w % NB == 0 and k_per_w >= NB

    @functools.partial(
        pl.kernel,
        mesh=mesh,
        out_type=jax.ShapeDtypeStruct((b, s, EMBED), jnp.float32),
        scratch_types=[
            pltpu.VMEM((k_per_w, idxc), jnp.int32),
            pltpu.VMEM((NB, idxc, EMBED), jnp.float32),
            pltpu.SemaphoreType.DMA((NB,)),
            pltpu.SemaphoreType.DMA((NB,)),
        ],
    )
    def k(table_hbm, idx_hbm, out_hbm, idx_v, rows_v, gsem, ssem):
        nc = 2
        wid = lax.axis_index("s") * nc + lax.axis_index("c")
        row_base = wid * rows_per_w
        pltpu.sync_copy(idx_hbm.at[wid], idx_v)

        def gather(j, slot):
            return pltpu.make_async_copy(
                table_hbm.at[idx_v.at[j]], rows_v.at[slot], gsem.at[slot])

        def stores(j, slot):
            return [
                pltpu.make_async_copy(
                    rows_v.at[slot, pl.ds(h * s, s)],
                    out_hbm.at[row_base + j * RPC + h],
                    ssem.at[slot])
                for h in range(RPC)
            ]

        for slot in range(GD):
            gather(slot, slot).start()

        def outer(i, _):
            g = i * NB
            for bslot in range(NB):
                j = g + bslot
                nslot = (bslot + GD) % NB
                @pl.when(j + GD - NB >= 0)
                def _():
                    for h_cp in stores(j + GD - NB, nslot):
                        h_cp.wait()

                @pl.when(j + GD < k_per_w)
                def _():
                    gather(j + GD, nslot).start()

                gather(j, bslot).wait()
                for h_cp in stores(j, bslot):
                    h_cp.start()
            return 0

        lax.fori_loop(0, k_per_w // NB, outer, 0)

        for j in range(k_per_w - SD, k_per_w):
            for h_cp in stores(j, j % NB):
                h_cp.wait()

    return k(table, idx3)


def kernel(cleavage_indices, pos_embed):
    b, s = cleavage_indices.shape
    info = plsc.get_sparse_core_info()
    nw = info.num_cores * info.num_subcores
    rows_per_w = b // nw
    k_per_w = rows_per_w // RPC
    pad = (-(s * RPC)) % 8
    idx3 = cleavage_indices.astype(jnp.int32).reshape(nw, k_per_w, s * RPC)
    idx3 = jnp.pad(idx3, ((0, 0), (0, 0), (0, pad)))
    out = _sc_gather(pos_embed, idx3, nw, k_per_w, b, s)
    return out
